# Initial kernel scaffold; baseline (speedup 1.0000x reference)
import jax
import jax.numpy as jnp
from jax.experimental import pallas as pl


def kernel(x, edge_index, W, b):
    raise NotImplementedError("write your pallas kernel here")


# speedup vs baseline: 14.0608x; 14.0608x over previous
import functools

import jax
import jax.numpy as jnp
from jax import lax
from jax.experimental import pallas as pl
from jax.experimental.pallas import tpu as pltpu
from jax.experimental.pallas import tpu_sc as plsc

NC = 2
NS = 16
LANES = 16


def _pad_rows(N):
    step = 128 * NS
    return ((N + step - 1) // step) * step

_f32 = jnp.float32
_i32 = jnp.int32


def _vsc_mesh():
    return plsc.VectorSubcoreMesh(core_axis_name="c", subcore_axis_name="s")


@functools.lru_cache(maxsize=None)
def _deg_kernel(N, E):
    EPT = E // (NC * NS)
    Np = _pad_rows(N)
    RPT = Np // NS
    assert RPT % LANES == 0 and EPT % LANES == 0
    HV = Np + LANES

    @functools.partial(
        pl.kernel,
        out_type=jax.ShapeDtypeStruct((NC, Np), _f32),
        mesh=_vsc_mesh(),
        scratch_types=[
            pltpu.VMEM((HV,), _f32),
            pltpu.VMEM((EPT,), _i32),
            pltpu.VMEM((RPT,), _f32),
            pltpu.VMEM((RPT,), _f32),
            pltpu.VMEM_SHARED((NS, Np), _f32),
        ],
    )
    def deg_k(row_hbm, out_hbm, hist_v, idx_v, comb_v, tmp_v, hist_sh):
        c = lax.axis_index("c")
        s = lax.axis_index("s")
        row0 = s * RPT

        def zfill(i, carry):
            hist_v[pl.ds(i * LANES, LANES)] = jnp.zeros((LANES,), _f32)
            return carry
        lax.fori_loop(0, HV // LANES, zfill, 0)

        pltpu.sync_copy(row_hbm.at[pl.ds((c * NS + s) * EPT, EPT)], idx_v)

        onehot0 = jnp.where(lax.iota(_i32, LANES) == 0, 1.0, 0.0).astype(_f32)

        def step(e, carry):
            iv = idx_v[pl.ds(e * LANES, LANES)]
            for l in range(LANES):
                i = iv[l]
                sl = pl.ds(i, LANES)
                hist_v[sl] = hist_v[sl] + onehot0
            return carry
        lax.fori_loop(0, EPT // LANES, step, 0)

        pltpu.sync_copy(hist_v.at[pl.ds(0, Np)], hist_sh.at[s])
        plsc.subcore_barrier()

        def zc(i, carry):
            comb_v[pl.ds(i * LANES, LANES)] = jnp.zeros((LANES,), _f32)
            return carry
        lax.fori_loop(0, RPT // LANES, zc, 0)

        def addtile(t, carry):
            pltpu.sync_copy(hist_sh.at[t, pl.ds(row0, RPT)], tmp_v)

            def vadd(i, carry2):
                sl = pl.ds(i * LANES, LANES)
                comb_v[sl] = comb_v[sl] + tmp_v[sl]
                return carry2
            lax.fori_loop(0, RPT // LANES, vadd, 0)
            return carry
        lax.fori_loop(0, NS, addtile, 0)

        pltpu.sync_copy(comb_v, out_hbm.at[c, pl.ds(row0, RPT)])

    return deg_k


@functools.lru_cache(maxsize=None)
def _agg_kernel(N, D, E):
    B = 80
    EPT = E // (NC * NS)
    Np = _pad_rows(N)
    RPT = Np // NS
    ZR = 128
    assert EPT % B == 0 and RPT % ZR == 0 and D % LANES == 0

    @functools.partial(
        pl.kernel,
        out_type=jax.ShapeDtypeStruct((NC * Np, D), _f32),
        mesh=_vsc_mesh(),
        scratch_types=[
            pltpu.VMEM((ZR, D), _f32),
            pltpu.VMEM((B, D), _f32),
            pltpu.VMEM((B,), _i32),
            pltpu.VMEM((B,), _i32),
            pltpu.VMEM_SHARED((Np, D), _f32),
            pltpu.SemaphoreType.DMA,
        ],
    )
    def agg_k(row_hbm, col_hbm, g_hbm, out_hbm,
              z_v, rows_v, idx_r, idx_c, acc_sh, sem):
        c = lax.axis_index("c")
        s = lax.axis_index("s")
        row0 = s * RPT
        nsub = D // LANES

        def zfill(k, carry):
            z_v[k // nsub, pl.ds((k % nsub) * LANES, LANES)] = (
                jnp.zeros((LANES,), _f32))
            return carry
        lax.fori_loop(0, ZR * nsub, zfill, 0)

        def zdma(j, carry):
            pltpu.sync_copy(z_v, acc_sh.at[pl.ds(row0 + j * ZR, ZR)])
            return carry
        lax.fori_loop(0, RPT // ZR, zdma, 0)

        plsc.subcore_barrier()

        base0 = c * (E // NC) + s * EPT

        def step(i, carry):
            b0 = base0 + i * B
            pltpu.sync_copy(row_hbm.at[pl.ds(b0, B)], idx_r)
            pltpu.sync_copy(col_hbm.at[pl.ds(b0, B)], idx_c)
            pltpu.async_copy(g_hbm.at[idx_r], rows_v, sem).wait()
            pltpu.sync_copy(rows_v, acc_sh.at[idx_c], add=True)
            return carry
        lax.fori_loop(0, EPT // B, step, 0)

        plsc.subcore_barrier()
        pltpu.sync_copy(acc_sh.at[pl.ds(row0, RPT)],
                        out_hbm.at[pl.ds(c * Np + row0, RPT)])

    return agg_k


def _inv_sqrt_deg(dt_ref):
    deg = dt_ref[:, 0:1] + dt_ref[:, 1:2]
    return jnp.where(deg > 0.0, lax.rsqrt(deg), 0.0)


def _prescale_body(x_ref, w_ref, dt_ref, g_ref):
    d = _inv_sqrt_deg(dt_ref)
    h = lax.dot_general(x_ref[...], w_ref[...],
                        (((1,), (1,)), ((), ())),
                        preferred_element_type=_f32)
    g_ref[...] = h * d


@functools.lru_cache(maxsize=None)
def _prescale_kernel(N, D_in, D_out, R=80):
    grid = N // R
    Np = _pad_rows(N)
    off = Np // R
    assert N % R == 0 and Np % R == 0
    return pl.pallas_call(
        _prescale_body,
        grid=(grid,),
        in_specs=[
            pl.BlockSpec((R, D_in), lambda i: (i, 0)),
            pl.BlockSpec((D_out, D_in), lambda i: (0, 0)),
            pl.BlockSpec((R, NC), lambda i: (i, 0)),
        ],
        out_specs=pl.BlockSpec((R, D_out), lambda i: (i, 0)),
        out_shape=jax.ShapeDtypeStruct((N, D_out), _f32),
    )


def _finish_body(pa_ref, pb_ref, dt_ref, b_ref, o_ref):
    d = _inv_sqrt_deg(dt_ref)
    o_ref[...] = (pa_ref[...] + pb_ref[...]) * d + b_ref[...]


@functools.lru_cache(maxsize=None)
def _finish_kernel(N, D, R=80):
    grid = N // R
    Np = _pad_rows(N)
    off = Np // R
    assert N % R == 0 and Np % R == 0
    return pl.pallas_call(
        _finish_body,
        grid=(grid,),
        in_specs=[
            pl.BlockSpec((R, D), lambda i: (i, 0)),
            pl.BlockSpec((R, D), lambda i: (i + off, 0)),
            pl.BlockSpec((R, NC), lambda i: (i, 0)),
            pl.BlockSpec((1, D), lambda i: (0, 0)),
        ],
        out_specs=pl.BlockSpec((R, D), lambda i: (i, 0)),
        out_shape=jax.ShapeDtypeStruct((N, D), _f32),
    )


def kernel(x, edge_index, W, b):
    N, D_in = x.shape
    D_out = W.shape[0]
    E = edge_index.shape[1]

    row = edge_index[0].astype(_i32)
    col = edge_index[1].astype(_i32)

    deg_p = _deg_kernel(N, E)(row)
    degT = jnp.transpose(deg_p)
    g = _prescale_kernel(N, D_in, D_out)(x, W, degT)
    p = _agg_kernel(N, D_out, E)(row, col, g)
    out = _finish_kernel(N, D_out)(p, p, degT,
                                   b.astype(_f32).reshape(1, D_out))
    return out


# speedup vs baseline: 30.3817x; 2.1607x over previous
import functools

import jax
import jax.numpy as jnp
from jax import lax
from jax.experimental import pallas as pl
from jax.experimental.pallas import tpu as pltpu
from jax.experimental.pallas import tpu_sc as plsc

NC = 2
NS = 16
LANES = 16


def _pad_rows(N):
    step = 128 * NS
    return ((N + step - 1) // step) * step

_f32 = jnp.float32
_i32 = jnp.int32


def _vsc_mesh():
    return plsc.VectorSubcoreMesh(core_axis_name="c", subcore_axis_name="s")


@functools.lru_cache(maxsize=None)
def _deg_kernel(N, E):
    EPT = E // (NC * NS)
    Np = _pad_rows(N)
    RPT = Np // NS
    assert RPT % LANES == 0 and EPT % LANES == 0
    HV = Np + LANES

    @functools.partial(
        pl.kernel,
        out_type=jax.ShapeDtypeStruct((NC, Np), _f32),
        mesh=_vsc_mesh(),
        scratch_types=[
            pltpu.VMEM((HV,), _f32),
            pltpu.VMEM((EPT,), _i32),
            pltpu.VMEM((RPT,), _f32),
            pltpu.VMEM((RPT,), _f32),
            pltpu.VMEM_SHARED((NS, Np), _f32),
        ],
    )
    def deg_k(row_hbm, out_hbm, hist_v, idx_v, comb_v, tmp_v, hist_sh):
        c = lax.axis_index("c")
        s = lax.axis_index("s")
        row0 = s * RPT

        def zfill(i, carry):
            hist_v[pl.ds(i * LANES, LANES)] = jnp.zeros((LANES,), _f32)
            return carry
        lax.fori_loop(0, HV // LANES, zfill, 0)

        pltpu.sync_copy(row_hbm.at[pl.ds((c * NS + s) * EPT, EPT)], idx_v)

        onehot0 = jnp.where(lax.iota(_i32, LANES) == 0, 1.0, 0.0).astype(_f32)

        def step(e, carry):
            iv = idx_v[pl.ds(e * LANES, LANES)]
            for l in range(LANES):
                i = iv[l]
                sl = pl.ds(i, LANES)
                hist_v[sl] = hist_v[sl] + onehot0
            return carry
        lax.fori_loop(0, EPT // LANES, step, 0)

        pltpu.sync_copy(hist_v.at[pl.ds(0, Np)], hist_sh.at[s])
        plsc.subcore_barrier()

        def zc(i, carry):
            comb_v[pl.ds(i * LANES, LANES)] = jnp.zeros((LANES,), _f32)
            return carry
        lax.fori_loop(0, RPT // LANES, zc, 0)

        def addtile(t, carry):
            pltpu.sync_copy(hist_sh.at[t, pl.ds(row0, RPT)], tmp_v)

            def vadd(i, carry2):
                sl = pl.ds(i * LANES, LANES)
                comb_v[sl] = comb_v[sl] + tmp_v[sl]
                return carry2
            lax.fori_loop(0, RPT // LANES, vadd, 0)
            return carry
        lax.fori_loop(0, NS, addtile, 0)

        pltpu.sync_copy(comb_v, out_hbm.at[c, pl.ds(row0, RPT)])

    return deg_k


@functools.lru_cache(maxsize=None)
def _agg_kernel(N, D, E):
    B = 80
    EPT = E // (NC * NS)
    Np = _pad_rows(N)
    RPT = Np // NS
    ZR = 128
    CH = EPT // B
    assert EPT % B == 0 and RPT % ZR == 0 and D % LANES == 0

    @functools.partial(
        pl.kernel,
        out_type=jax.ShapeDtypeStruct((NC, Np, D), _f32),
        mesh=_vsc_mesh(),
        scratch_types=[
            pltpu.VMEM((ZR, D), _f32),
            pltpu.VMEM((2, B, D), _f32),
            pltpu.VMEM((EPT,), _i32),
            pltpu.VMEM((2, B), _i32),
            pltpu.VMEM_SHARED((Np, D), _f32),
            pltpu.SemaphoreType.DMA,
            pltpu.SemaphoreType.DMA,
            pltpu.SemaphoreType.DMA,
            pltpu.SemaphoreType.DMA,
        ],
    )
    def agg_k(row_hbm, col_hbm, g_hbm, out_hbm,
              z_v, rows_v, idxr_v, idxc_v, acc_sh, sg0, sg1, sc0, sc1):
        c = lax.axis_index("c")
        s = lax.axis_index("s")
        row0 = s * RPT
        nsub = D // LANES
        sgs = (sg0, sg1)
        scs = (sc0, sc1)

        def zfill(k, carry):
            z_v[k // nsub, pl.ds((k % nsub) * LANES, LANES)] = (
                jnp.zeros((LANES,), _f32))
            return carry
        lax.fori_loop(0, ZR * nsub, zfill, 0)

        def zdma(j, carry):
            pltpu.sync_copy(z_v, acc_sh.at[pl.ds(row0 + j * ZR, ZR)])
            return carry
        lax.fori_loop(0, RPT // ZR, zdma, 0)

        base0 = c * (E // NC) + s * EPT
        pltpu.sync_copy(row_hbm.at[pl.ds(base0, EPT)], idxr_v)

        plsc.subcore_barrier()

        def issue_gather(j, b):
            return pltpu.async_copy(
                g_hbm.at[idxr_v.at[pl.ds(j * B, B)]], rows_v.at[b], sgs[b])

        def issue_idxc(j, b):
            return pltpu.async_copy(
                col_hbm.at[pl.ds(base0 + j * B, B)], idxc_v.at[b], scs[b])

        issue_gather(0, 0)
        issue_idxc(0, 0)
        issue_gather(1, 1)
        issue_idxc(1, 1)

        def pair(jj, carry):
            for b in range(2):
                j = jj * 2 + b

                @pl.when(j < CH)
                def _():
                    pltpu.make_async_copy(
                        g_hbm.at[idxr_v.at[pl.ds(j * B, B)]],
                        rows_v.at[b], sgs[b]).wait()
                    pltpu.make_async_copy(
                        col_hbm.at[pl.ds(base0 + j * B, B)],
                        idxc_v.at[b], scs[b]).wait()
                    pltpu.sync_copy(rows_v.at[b],
                                    acc_sh.at[idxc_v.at[b]], add=True)

                    @pl.when(j + 2 < CH)
                    def _():
                        issue_gather(j + 2, b)
                        issue_idxc(j + 2, b)
            return carry
        lax.fori_loop(0, (CH + 1) // 2, pair, 0)

        plsc.subcore_barrier()
        pltpu.sync_copy(acc_sh.at[pl.ds(row0, RPT)],
                        out_hbm.at[c, pl.ds(row0, RPT)])

    return agg_k


def _inv_sqrt_deg(dt_ref):
    deg = dt_ref[:, 0:1] + dt_ref[:, 1:2]
    return jnp.where(deg > 0.0, lax.rsqrt(deg), 0.0)


def _prescale_body(x_ref, w_ref, dt_ref, g_ref):
    d = _inv_sqrt_deg(dt_ref)
    h = lax.dot_general(x_ref[...], w_ref[...],
                        (((1,), (1,)), ((), ())),
                        preferred_element_type=_f32)
    g_ref[...] = h * d


@functools.lru_cache(maxsize=None)
def _prescale_kernel(N, D_in, D_out, R=400):
    grid = N // R
    assert N % R == 0
    return pl.pallas_call(
        _prescale_body,
        grid=(grid,),
        in_specs=[
            pl.BlockSpec((R, D_in), lambda i: (i, 0)),
            pl.BlockSpec((D_out, D_in), lambda i: (0, 0)),
            pl.BlockSpec((R, NC), lambda i: (i, 0)),
        ],
        out_specs=pl.BlockSpec((R, D_out), lambda i: (i, 0)),
        out_shape=jax.ShapeDtypeStruct((N, D_out), _f32),
    )


def _finish_body(p_ref, dt_ref, b_ref, o_ref):
    d = _inv_sqrt_deg(dt_ref)
    o_ref[...] = (p_ref[0] + p_ref[1]) * d + b_ref[...]


@functools.lru_cache(maxsize=None)
def _finish_kernel(N, D, R=400):
    grid = N // R
    assert N % R == 0
    return pl.pallas_call(
        _finish_body,
        grid=(grid,),
        in_specs=[
            pl.BlockSpec((NC, R, D), lambda i: (0, i, 0)),
            pl.BlockSpec((R, NC), lambda i: (i, 0)),
            pl.BlockSpec((1, D), lambda i: (0, 0)),
        ],
        out_specs=pl.BlockSpec((R, D), lambda i: (i, 0)),
        out_shape=jax.ShapeDtypeStruct((N, D), _f32),
    )


def kernel(x, edge_index, W, b):
    N, D_in = x.shape
    D_out = W.shape[0]
    E = edge_index.shape[1]

    row = edge_index[0].astype(_i32)
    col = edge_index[1].astype(_i32)

    deg_p = _deg_kernel(N, E)(row)
    degT = jnp.transpose(deg_p)
    g = _prescale_kernel(N, D_in, D_out)(x, W, degT)
    p = _agg_kernel(N, D_out, E)(row, col, g)
    out = _finish_kernel(N, D_out)(p, degT,
                                   b.astype(_f32).reshape(1, D_out))
    return out


# speedup vs baseline: 32.3016x; 1.0632x over previous
import functools

import jax
import jax.numpy as jnp
from jax import lax
from jax.experimental import pallas as pl
from jax.experimental.pallas import tpu as pltpu
from jax.experimental.pallas import tpu_sc as plsc

NC = 2
NS = 16
LANES = 16


def _pad_rows(N):
    step = 128 * NS
    return ((N + step - 1) // step) * step

_f32 = jnp.float32
_i32 = jnp.int32


def _vsc_mesh():
    return plsc.VectorSubcoreMesh(core_axis_name="c", subcore_axis_name="s")


@functools.lru_cache(maxsize=None)
def _deg_kernel(N, E):
    EPT = E // (NC * NS)
    Np = _pad_rows(N)
    RPT = Np // NS
    NQ = 5
    QL = EPT // NQ
    assert RPT % LANES == 0 and QL % LANES == 0 and EPT % NQ == 0
    HV = Np + LANES

    @functools.partial(
        pl.kernel,
        out_type=jax.ShapeDtypeStruct((NC, Np), _f32),
        mesh=_vsc_mesh(),
        scratch_types=(
            [pltpu.VMEM((HV,), _f32)] * NQ + [
                pltpu.VMEM((EPT,), _i32),
                pltpu.VMEM((RPT,), _f32),
                pltpu.VMEM((RPT,), _f32),
                pltpu.VMEM_SHARED((NS, Np), _f32),
            ]
        ),
    )
    def deg_k(row_hbm, out_hbm, *refs):
        hq = refs[:NQ]
        idx_v, comb_v, tmp_v, hist_sh = refs[NQ:]
        c = lax.axis_index("c")
        s = lax.axis_index("s")
        row0 = s * RPT

        def zfill(i, carry):
            for q in range(NQ):
                hq[q][pl.ds(i * LANES, LANES)] = jnp.zeros((LANES,), _f32)
            return carry
        lax.fori_loop(0, HV // LANES, zfill, 0)

        pltpu.sync_copy(row_hbm.at[pl.ds((c * NS + s) * EPT, EPT)], idx_v)

        onehot0 = jnp.where(lax.iota(_i32, LANES) == 0, 1.0, 0.0).astype(_f32)

        def step(e, carry):
            ivs = [idx_v[pl.ds(q * QL + e * LANES, LANES)] for q in range(NQ)]
            for l in range(LANES):
                for q in range(NQ):
                    i = ivs[q][l]
                    sl = pl.ds(i, LANES)
                    hq[q][sl] = hq[q][sl] + onehot0
            return carry
        lax.fori_loop(0, QL // LANES, step, 0)

        def merge(i, carry):
            sl = pl.ds(i * LANES, LANES)
            acc = hq[0][sl]
            for q in range(1, NQ):
                acc = acc + hq[q][sl]
            hq[0][sl] = acc
            return carry
        lax.fori_loop(0, Np // LANES, merge, 0)

        pltpu.sync_copy(hq[0].at[pl.ds(0, Np)], hist_sh.at[s])
        plsc.subcore_barrier()

        def zc(i, carry):
            comb_v[pl.ds(i * LANES, LANES)] = jnp.zeros((LANES,), _f32)
            return carry
        lax.fori_loop(0, RPT // LANES, zc, 0)

        def addtile(t, carry):
            pltpu.sync_copy(hist_sh.at[t, pl.ds(row0, RPT)], tmp_v)

            def vadd(i, carry2):
                sl = pl.ds(i * LANES, LANES)
                comb_v[sl] = comb_v[sl] + tmp_v[sl]
                return carry2
            lax.fori_loop(0, RPT // LANES, vadd, 0)
            return carry
        lax.fori_loop(0, NS, addtile, 0)

        pltpu.sync_copy(comb_v, out_hbm.at[c, pl.ds(row0, RPT)])

    return deg_k


@functools.lru_cache(maxsize=None)
def _agg_kernel(N, D, E):
    B = 128
    EPT = E // (NC * NS)
    Np = _pad_rows(N)
    RPT = Np // NS
    ZR = 32
    CH = EPT // B
    TB = EPT - CH * B
    assert TB % 8 == 0 and RPT % ZR == 0 and D % LANES == 0

    @functools.partial(
        pl.kernel,
        out_type=jax.ShapeDtypeStruct((NC, Np, D), _f32),
        mesh=_vsc_mesh(),
        scratch_types=[
            pltpu.VMEM((ZR, D), _f32),
            pltpu.VMEM((2, B, D), _f32),
            pltpu.VMEM((EPT,), _i32),
            pltpu.VMEM((2, B), _i32),
            pltpu.VMEM((max(TB, 8),), _i32),
            pltpu.VMEM_SHARED((Np, D), _f32),
            pltpu.SemaphoreType.DMA,
            pltpu.SemaphoreType.DMA,
            pltpu.SemaphoreType.DMA,
            pltpu.SemaphoreType.DMA,
        ],
    )
    def agg_k(row_hbm, col_hbm, g_hbm, out_hbm,
              z_v, rows_v, idxr_v, idxc_v, idxt_v, acc_sh,
              sg0, sg1, sc0, sc1):
        c = lax.axis_index("c")
        s = lax.axis_index("s")
        row0 = s * RPT
        nsub = D // LANES
        sgs = (sg0, sg1)
        scs = (sc0, sc1)

        def zfill(k, carry):
            z_v[k // nsub, pl.ds((k % nsub) * LANES, LANES)] = (
                jnp.zeros((LANES,), _f32))
            return carry
        lax.fori_loop(0, ZR * nsub, zfill, 0)

        def zdma(j, carry):
            pltpu.sync_copy(z_v, acc_sh.at[pl.ds(row0 + j * ZR, ZR)])
            return carry
        lax.fori_loop(0, RPT // ZR, zdma, 0)

        base0 = c * (E // NC) + s * EPT
        pltpu.sync_copy(row_hbm.at[pl.ds(base0, EPT)], idxr_v)

        plsc.subcore_barrier()

        def issue_gather(j, b):
            return pltpu.async_copy(
                g_hbm.at[idxr_v.at[pl.ds(j * B, B)]], rows_v.at[b], sgs[b])

        def issue_idxc(j, b):
            return pltpu.async_copy(
                col_hbm.at[pl.ds(base0 + j * B, B)], idxc_v.at[b], scs[b])

        issue_gather(0, 0)
        issue_idxc(0, 0)
        issue_gather(1, 1)
        issue_idxc(1, 1)

        def pair(jj, carry):
            for b in range(2):
                j = jj * 2 + b

                @pl.when(j < CH)
                def _():
                    pltpu.make_async_copy(
                        g_hbm.at[idxr_v.at[pl.ds(j * B, B)]],
                        rows_v.at[b], sgs[b]).wait()
                    pltpu.make_async_copy(
                        col_hbm.at[pl.ds(base0 + j * B, B)],
                        idxc_v.at[b], scs[b]).wait()
                    pltpu.sync_copy(rows_v.at[b],
                                    acc_sh.at[idxc_v.at[b]], add=True)

                    @pl.when(j + 2 < CH)
                    def _():
                        issue_gather(j + 2, b)
                        issue_idxc(j + 2, b)
            return carry
        lax.fori_loop(0, (CH + 1) // 2, pair, 0)

        if TB:
            t0 = CH * B
            pltpu.sync_copy(col_hbm.at[pl.ds(base0 + t0, TB)], idxt_v)
            pltpu.async_copy(
                g_hbm.at[idxr_v.at[pl.ds(t0, TB)]],
                rows_v.at[0, pl.ds(0, TB)], sg0).wait()
            pltpu.sync_copy(rows_v.at[0, pl.ds(0, TB)],
                            acc_sh.at[idxt_v], add=True)

        plsc.subcore_barrier()
        pltpu.sync_copy(acc_sh.at[pl.ds(row0, RPT)],
                        out_hbm.at[c, pl.ds(row0, RPT)])

    return agg_k


def _inv_sqrt_deg(dt_ref):
    deg = dt_ref[:, 0:1] + dt_ref[:, 1:2]
    return jnp.where(deg > 0.0, lax.rsqrt(deg), 0.0)


def _prescale_body(x_ref, w_ref, dt_ref, g_ref):
    d = _inv_sqrt_deg(dt_ref)
    h = lax.dot_general(x_ref[...], w_ref[...],
                        (((1,), (1,)), ((), ())),
                        preferred_element_type=_f32)
    g_ref[...] = h * d


@functools.lru_cache(maxsize=None)
def _prescale_kernel(N, D_in, D_out, R=400):
    grid = N // R
    assert N % R == 0
    return pl.pallas_call(
        _prescale_body,
        grid=(grid,),
        in_specs=[
            pl.BlockSpec((R, D_in), lambda i: (i, 0)),
            pl.BlockSpec((D_out, D_in), lambda i: (0, 0)),
            pl.BlockSpec((R, NC), lambda i: (i, 0)),
        ],
        out_specs=pl.BlockSpec((R, D_out), lambda i: (i, 0)),
        out_shape=jax.ShapeDtypeStruct((N, D_out), _f32),
    )


def _finish_body(p_ref, dt_ref, b_ref, o_ref):
    d = _inv_sqrt_deg(dt_ref)
    o_ref[...] = (p_ref[0] + p_ref[1]) * d + b_ref[...]


@functools.lru_cache(maxsize=None)
def _finish_kernel(N, D, R=400):
    grid = N // R
    assert N % R == 0
    return pl.pallas_call(
        _finish_body,
        grid=(grid,),
        in_specs=[
            pl.BlockSpec((NC, R, D), lambda i: (0, i, 0)),
            pl.BlockSpec((R, NC), lambda i: (i, 0)),
            pl.BlockSpec((1, D), lambda i: (0, 0)),
        ],
        out_specs=pl.BlockSpec((R, D), lambda i: (i, 0)),
        out_shape=jax.ShapeDtypeStruct((N, D), _f32),
    )


def kernel(x, edge_index, W, b):
    N, D_in = x.shape
    D_out = W.shape[0]
    E = edge_index.shape[1]

    row = edge_index[0].astype(_i32)
    col = edge_index[1].astype(_i32)

    deg_p = _deg_kernel(N, E)(row)
    degT = jnp.transpose(deg_p)
    g = _prescale_kernel(N, D_in, D_out)(x, W, degT)
    p = _agg_kernel(N, D_out, E)(row, col, g)
    out = _finish_kernel(N, D_out)(p, degT,
                                   b.astype(_f32).reshape(1, D_out))
    return out


# speedup vs baseline: 35.5160x; 1.0995x over previous
import functools

import jax
import jax.numpy as jnp
from jax import lax
from jax.experimental import pallas as pl
from jax.experimental.pallas import tpu as pltpu
from jax.experimental.pallas import tpu_sc as plsc

NC = 2
NS = 16
LANES = 16


def _pad_rows(N):
    step = 128 * NS
    return ((N + step - 1) // step) * step

_f32 = jnp.float32
_i32 = jnp.int32


def _vsc_mesh():
    return plsc.VectorSubcoreMesh(core_axis_name="c", subcore_axis_name="s")


@functools.lru_cache(maxsize=None)
def _deg_kernel(N, E):
    EPT = E // (NC * NS)
    Np = _pad_rows(N)
    RPT = Np // NS
    NQ = 5
    QL = EPT // NQ
    assert RPT % LANES == 0 and QL % LANES == 0 and EPT % NQ == 0
    HV = Np + LANES

    @functools.partial(
        pl.kernel,
        out_type=jax.ShapeDtypeStruct((NC, Np), _f32),
        mesh=_vsc_mesh(),
        scratch_types=(
            [pltpu.VMEM((HV,), _f32)] * NQ + [
                pltpu.VMEM((EPT,), _i32),
                pltpu.VMEM((RPT,), _f32),
                pltpu.VMEM((RPT,), _f32),
                pltpu.VMEM_SHARED((NS, Np), _f32),
            ]
        ),
    )
    def deg_k(ei_hbm, out_hbm, *refs):
        hq = refs[:NQ]
        idx_v, comb_v, tmp_v, hist_sh = refs[NQ:]
        c = lax.axis_index("c")
        s = lax.axis_index("s")
        row0 = s * RPT

        def zfill(i, carry):
            for q in range(NQ):
                hq[q][pl.ds(i * LANES, LANES)] = jnp.zeros((LANES,), _f32)
            return carry
        lax.fori_loop(0, HV // LANES, zfill, 0)

        pltpu.sync_copy(ei_hbm.at[pl.ds((c * NS + s) * EPT, EPT)], idx_v)

        onehot0 = jnp.where(lax.iota(_i32, LANES) == 0, 1.0, 0.0).astype(_f32)

        def step(e, carry):
            ivs = [idx_v[pl.ds(q * QL + e * LANES, LANES)] for q in range(NQ)]
            for l in range(LANES):
                for q in range(NQ):
                    i = ivs[q][l]
                    sl = pl.ds(i, LANES)
                    hq[q][sl] = hq[q][sl] + onehot0
            return carry
        lax.fori_loop(0, QL // LANES, step, 0)

        def merge(i, carry):
            sl = pl.ds(i * LANES, LANES)
            acc = hq[0][sl]
            for q in range(1, NQ):
                acc = acc + hq[q][sl]
            hq[0][sl] = acc
            return carry
        lax.fori_loop(0, Np // LANES, merge, 0)

        pltpu.sync_copy(hq[0].at[pl.ds(0, Np)], hist_sh.at[s])
        plsc.subcore_barrier()

        def zc(i, carry):
            comb_v[pl.ds(i * LANES, LANES)] = jnp.zeros((LANES,), _f32)
            return carry
        lax.fori_loop(0, RPT // LANES, zc, 0)

        def addtile(t, carry):
            pltpu.sync_copy(hist_sh.at[t, pl.ds(row0, RPT)], tmp_v)

            def vadd(i, carry2):
                sl = pl.ds(i * LANES, LANES)
                comb_v[sl] = comb_v[sl] + tmp_v[sl]
                return carry2
            lax.fori_loop(0, RPT // LANES, vadd, 0)
            return carry
        lax.fori_loop(0, NS, addtile, 0)

        pltpu.sync_copy(comb_v, out_hbm.at[c, pl.ds(row0, RPT)])

    return deg_k


@functools.lru_cache(maxsize=None)
def _agg_kernel(N, D, E):
    B = 64
    DEPTH = 4
    EPT = E // (NC * NS)
    Np = _pad_rows(N)
    RPT = Np // NS
    ZR = 8
    CH = EPT // B
    TB = EPT - CH * B
    assert TB % 8 == 0 and RPT % ZR == 0 and D % LANES == 0

    @functools.partial(
        pl.kernel,
        out_type=jax.ShapeDtypeStruct((NC, Np, D), _f32),
        mesh=_vsc_mesh(),
        scratch_types=(
            [pltpu.VMEM((ZR, D), _f32),
             pltpu.VMEM((DEPTH, B, D), _f32),
             pltpu.VMEM((EPT,), _i32),
             pltpu.VMEM((DEPTH, B), _i32),
             pltpu.VMEM((max(TB, 8),), _i32),
             pltpu.VMEM_SHARED((Np, D), _f32)]
            + [pltpu.SemaphoreType.DMA] * (2 * DEPTH)
        ),
    )
    def agg_k(ei_hbm, g_hbm, out_hbm,
              z_v, rows_v, idxr_v, idxc_v, idxt_v, acc_sh, *sems):
        c = lax.axis_index("c")
        s = lax.axis_index("s")
        row0 = s * RPT
        nsub = D // LANES
        sgs = sems[:DEPTH]
        scs = sems[DEPTH:]

        def zfill(k, carry):
            z_v[k // nsub, pl.ds((k % nsub) * LANES, LANES)] = (
                jnp.zeros((LANES,), _f32))
            return carry
        lax.fori_loop(0, ZR * nsub, zfill, 0)

        def zdma(j, carry):
            pltpu.sync_copy(z_v, acc_sh.at[pl.ds(row0 + j * ZR, ZR)])
            return carry
        lax.fori_loop(0, RPT // ZR, zdma, 0)

        base0 = c * (E // NC) + s * EPT
        pltpu.sync_copy(ei_hbm.at[pl.ds(base0, EPT)], idxr_v)

        plsc.subcore_barrier()

        def issue_gather(j, b):
            return pltpu.async_copy(
                g_hbm.at[idxr_v.at[pl.ds(j * B, B)]], rows_v.at[b], sgs[b])

        def issue_idxc(j, b):
            return pltpu.async_copy(
                ei_hbm.at[pl.ds(E + base0 + j * B, B)], idxc_v.at[b], scs[b])

        for d0 in range(DEPTH):
            issue_gather(d0, d0)
            issue_idxc(d0, d0)

        def ring(jj, carry):
            for b in range(DEPTH):
                j = jj * DEPTH + b

                @pl.when(j < CH)
                def _():
                    pltpu.make_async_copy(
                        g_hbm.at[idxr_v.at[pl.ds(j * B, B)]],
                        rows_v.at[b], sgs[b]).wait()
                    pltpu.make_async_copy(
                        ei_hbm.at[pl.ds(E + base0 + j * B, B)],
                        idxc_v.at[b], scs[b]).wait()
                    pltpu.sync_copy(rows_v.at[b],
                                    acc_sh.at[idxc_v.at[b]], add=True)

                    @pl.when(j + DEPTH < CH)
                    def _():
                        issue_gather(j + DEPTH, b)
                        issue_idxc(j + DEPTH, b)
            return carry
        lax.fori_loop(0, (CH + DEPTH - 1) // DEPTH, ring, 0)

        if TB:
            t0 = CH * B
            pltpu.sync_copy(ei_hbm.at[pl.ds(E + base0 + t0, TB)], idxt_v)
            pltpu.async_copy(
                g_hbm.at[idxr_v.at[pl.ds(t0, TB)]],
                rows_v.at[0, pl.ds(0, TB)], sgs[0]).wait()
            pltpu.sync_copy(rows_v.at[0, pl.ds(0, TB)],
                            acc_sh.at[idxt_v], add=True)

        plsc.subcore_barrier()
        pltpu.sync_copy(acc_sh.at[pl.ds(row0, RPT)],
                        out_hbm.at[c, pl.ds(row0, RPT)])

    return agg_k


def _inv_sqrt_deg(dt_ref):
    deg = dt_ref[:, 0:1] + dt_ref[:, 1:2]
    return jnp.where(deg > 0.0, lax.rsqrt(deg), 0.0)


def _prescale_body(x_ref, w_ref, dt_ref, g_ref):
    d = _inv_sqrt_deg(dt_ref)
    h = lax.dot_general(x_ref[...], w_ref[...],
                        (((1,), (1,)), ((), ())),
                        preferred_element_type=_f32)
    g_ref[...] = h * d


@functools.lru_cache(maxsize=None)
def _prescale_kernel(N, D_in, D_out, R=400):
    grid = N // R
    assert N % R == 0
    return pl.pallas_call(
        _prescale_body,
        grid=(grid,),
        in_specs=[
            pl.BlockSpec((R, D_in), lambda i: (i, 0)),
            pl.BlockSpec((D_out, D_in), lambda i: (0, 0)),
            pl.BlockSpec((R, NC), lambda i: (i, 0)),
        ],
        out_specs=pl.BlockSpec((R, D_out), lambda i: (i, 0)),
        out_shape=jax.ShapeDtypeStruct((N, D_out), _f32),
    )


def _finish_body(p_ref, dt_ref, b_ref, o_ref):
    d = _inv_sqrt_deg(dt_ref)
    o_ref[...] = (p_ref[0] + p_ref[1]) * d + b_ref[...]


@functools.lru_cache(maxsize=None)
def _finish_kernel(N, D, R=400):
    grid = N // R
    assert N % R == 0
    return pl.pallas_call(
        _finish_body,
        grid=(grid,),
        in_specs=[
            pl.BlockSpec((NC, R, D), lambda i: (0, i, 0)),
            pl.BlockSpec((R, NC), lambda i: (i, 0)),
            pl.BlockSpec((1, D), lambda i: (0, 0)),
        ],
        out_specs=pl.BlockSpec((R, D), lambda i: (i, 0)),
        out_shape=jax.ShapeDtypeStruct((N, D), _f32),
    )


def kernel(x, edge_index, W, b):
    N, D_in = x.shape
    D_out = W.shape[0]
    E = edge_index.shape[1]

    ei = edge_index.astype(_i32).reshape(2 * E)

    deg_p = _deg_kernel(N, E)(ei)
    degT = jnp.transpose(deg_p)
    g = _prescale_kernel(N, D_in, D_out)(x, W, degT)
    p = _agg_kernel(N, D_out, E)(ei, g)
    out = _finish_kernel(N, D_out)(p, degT,
                                   b.astype(_f32).reshape(1, D_out))
    return out


# speedup vs baseline: 37.2622x; 1.0492x over previous
import functools

import jax
import jax.numpy as jnp
from jax import lax
from jax.experimental import pallas as pl
from jax.experimental.pallas import tpu as pltpu
from jax.experimental.pallas import tpu_sc as plsc

NC = 2
NS = 16
LANES = 16


def _pad_rows(N):
    step = 128 * NS
    return ((N + step - 1) // step) * step

_f32 = jnp.float32
_i32 = jnp.int32


def _vsc_mesh():
    return plsc.VectorSubcoreMesh(core_axis_name="c", subcore_axis_name="s")


@functools.lru_cache(maxsize=None)
def _deg_kernel(N, E):
    EPT = E // (NC * NS)
    Np = _pad_rows(N)
    RPT = Np // NS
    NQ = 5
    QL = EPT // NQ
    assert RPT % LANES == 0 and QL % LANES == 0 and EPT % NQ == 0
    HV = Np + LANES

    @functools.partial(
        pl.kernel,
        out_type=jax.ShapeDtypeStruct((NC, Np), _f32),
        mesh=_vsc_mesh(),
        scratch_types=(
            [pltpu.VMEM((HV,), _f32)] * NQ + [
                pltpu.VMEM((EPT,), _i32),
                pltpu.VMEM((RPT,), _f32),
                pltpu.VMEM((RPT,), _f32),
                pltpu.VMEM_SHARED((NS, Np), _f32),
            ]
        ),
    )
    def deg_k(ei_hbm, out_hbm, *refs):
        hq = refs[:NQ]
        idx_v, comb_v, tmp_v, hist_sh = refs[NQ:]
        c = lax.axis_index("c")
        s = lax.axis_index("s")
        row0 = s * RPT

        def zfill(i, carry):
            for q in range(NQ):
                hq[q][pl.ds(i * LANES, LANES)] = jnp.zeros((LANES,), _f32)
            return carry
        lax.fori_loop(0, HV // LANES, zfill, 0)

        pltpu.sync_copy(ei_hbm.at[pl.ds((c * NS + s) * EPT, EPT)], idx_v)

        onehot0 = jnp.where(lax.iota(_i32, LANES) == 0, 1.0, 0.0).astype(_f32)

        def step(e, carry):
            ivs = [idx_v[pl.ds(q * QL + e * LANES, LANES)] for q in range(NQ)]
            for l in range(LANES):
                for q in range(NQ):
                    i = ivs[q][l]
                    sl = pl.ds(i, LANES)
                    hq[q][sl] = hq[q][sl] + onehot0
            return carry
        lax.fori_loop(0, QL // LANES, step, 0)

        def merge(i, carry):
            sl = pl.ds(i * LANES, LANES)
            acc = hq[0][sl]
            for q in range(1, NQ):
                acc = acc + hq[q][sl]
            hq[0][sl] = acc
            return carry
        lax.fori_loop(0, Np // LANES, merge, 0)

        pltpu.sync_copy(hq[0].at[pl.ds(0, Np)], hist_sh.at[s])
        plsc.subcore_barrier()

        def zc(i, carry):
            comb_v[pl.ds(i * LANES, LANES)] = jnp.zeros((LANES,), _f32)
            return carry
        lax.fori_loop(0, RPT // LANES, zc, 0)

        def addtile(t, carry):
            pltpu.sync_copy(hist_sh.at[t, pl.ds(row0, RPT)], tmp_v)

            def vadd(i, carry2):
                sl = pl.ds(i * LANES, LANES)
                comb_v[sl] = comb_v[sl] + tmp_v[sl]
                return carry2
            lax.fori_loop(0, RPT // LANES, vadd, 0)
            return carry
        lax.fori_loop(0, NS, addtile, 0)

        pltpu.sync_copy(comb_v, out_hbm.at[c, pl.ds(row0, RPT)])

    return deg_k


@functools.lru_cache(maxsize=None)
def _agg_kernel(N, D, E):
    B = 64
    DEPTH = 4
    EPT = E // (NC * NS)
    Np = _pad_rows(N)
    RPT = Np // NS
    ZR = 8
    CH = EPT // B
    TB = EPT - CH * B
    assert TB % 8 == 0 and RPT % ZR == 0 and D % LANES == 0

    @functools.partial(
        pl.kernel,
        out_type=jax.ShapeDtypeStruct((NC, Np, D), _f32),
        mesh=_vsc_mesh(),
        scratch_types=(
            [pltpu.VMEM((ZR, D), _f32),
             pltpu.VMEM((DEPTH, B, D), _f32),
             pltpu.VMEM((EPT,), _i32),
             pltpu.VMEM((DEPTH, B), _i32),
             pltpu.VMEM((max(TB, 8),), _i32),
             pltpu.VMEM_SHARED((Np, D), _f32)]
            + [pltpu.SemaphoreType.DMA] * (2 * DEPTH)
        ),
    )
    def agg_k(ei_hbm, g_hbm, out_hbm,
              z_v, rows_v, idxr_v, idxc_v, idxt_v, acc_sh, *sems):
        c = lax.axis_index("c")
        s = lax.axis_index("s")
        row0 = s * RPT
        nsub = D // LANES
        sgs = sems[:DEPTH]
        scs = sems[DEPTH:]

        def zfill(k, carry):
            z_v[k // nsub, pl.ds((k % nsub) * LANES, LANES)] = (
                jnp.zeros((LANES,), _f32))
            return carry
        lax.fori_loop(0, ZR * nsub, zfill, 0)

        def zdma(j, carry):
            pltpu.sync_copy(z_v, acc_sh.at[pl.ds(row0 + j * ZR, ZR)])
            return carry
        lax.fori_loop(0, RPT // ZR, zdma, 0)

        base0 = c * (E // NC) + s * EPT
        pltpu.sync_copy(ei_hbm.at[pl.ds(base0, EPT)], idxr_v)

        plsc.subcore_barrier()

        def issue_gather(j, b):
            return pltpu.async_copy(
                g_hbm.at[idxr_v.at[pl.ds(j * B, B)]], rows_v.at[b], sgs[b])

        def issue_idxc(j, b):
            return pltpu.async_copy(
                ei_hbm.at[pl.ds(E + base0 + j * B, B)], idxc_v.at[b], scs[b])

        for d0 in range(DEPTH):
            issue_gather(d0, d0)
            issue_idxc(d0, d0)

        def ring(jj, carry):
            for b in range(DEPTH):
                j = jj * DEPTH + b

                @pl.when(j < CH)
                def _():
                    pltpu.make_async_copy(
                        g_hbm.at[idxr_v.at[pl.ds(j * B, B)]],
                        rows_v.at[b], sgs[b]).wait()
                    pltpu.make_async_copy(
                        ei_hbm.at[pl.ds(E + base0 + j * B, B)],
                        idxc_v.at[b], scs[b]).wait()
                    pltpu.sync_copy(rows_v.at[b],
                                    acc_sh.at[idxc_v.at[b]], add=True)

                    @pl.when(j + DEPTH < CH)
                    def _():
                        issue_gather(j + DEPTH, b)
                        issue_idxc(j + DEPTH, b)
            return carry
        lax.fori_loop(0, (CH + DEPTH - 1) // DEPTH, ring, 0)

        if TB:
            t0 = CH * B
            pltpu.sync_copy(ei_hbm.at[pl.ds(E + base0 + t0, TB)], idxt_v)
            pltpu.async_copy(
                g_hbm.at[idxr_v.at[pl.ds(t0, TB)]],
                rows_v.at[0, pl.ds(0, TB)], sgs[0]).wait()
            pltpu.sync_copy(rows_v.at[0, pl.ds(0, TB)],
                            acc_sh.at[idxt_v], add=True)

        plsc.subcore_barrier()
        pltpu.sync_copy(acc_sh.at[pl.ds(row0, RPT)],
                        out_hbm.at[c, pl.ds(row0, RPT)])

    return agg_k


def _inv_sqrt_deg(dp_ref):
    deg = dp_ref[0:1, :] + dp_ref[1:2, :]
    d = jnp.where(deg > 0.0, lax.rsqrt(deg), 0.0)
    return jnp.transpose(d)


def _prescale_body(x_ref, w_ref, dt_ref, g_ref):
    d = _inv_sqrt_deg(dt_ref)
    h = lax.dot_general(x_ref[...], w_ref[...],
                        (((1,), (1,)), ((), ())),
                        preferred_element_type=_f32)
    g_ref[...] = h * d


@functools.lru_cache(maxsize=None)
def _prescale_kernel(N, D_in, D_out, R=512):
    grid = (N + R - 1) // R
    assert grid * R <= _pad_rows(N)
    return pl.pallas_call(
        _prescale_body,
        grid=(grid,),
        in_specs=[
            pl.BlockSpec((R, D_in), lambda i: (i, 0)),
            pl.BlockSpec((D_out, D_in), lambda i: (0, 0)),
            pl.BlockSpec((NC, R), lambda i: (0, i)),
        ],
        out_specs=pl.BlockSpec((R, D_out), lambda i: (i, 0)),
        out_shape=jax.ShapeDtypeStruct((N, D_out), _f32),
    )


def _finish_body(p_ref, dt_ref, b_ref, o_ref):
    d = _inv_sqrt_deg(dt_ref)
    o_ref[...] = (p_ref[0] + p_ref[1]) * d + b_ref[...]


@functools.lru_cache(maxsize=None)
def _finish_kernel(N, D, R=512):
    grid = (N + R - 1) // R
    assert grid * R <= _pad_rows(N)
    return pl.pallas_call(
        _finish_body,
        grid=(grid,),
        in_specs=[
            pl.BlockSpec((NC, R, D), lambda i: (0, i, 0)),
            pl.BlockSpec((NC, R), lambda i: (0, i)),
            pl.BlockSpec((1, D), lambda i: (0, 0)),
        ],
        out_specs=pl.BlockSpec((R, D), lambda i: (i, 0)),
        out_shape=jax.ShapeDtypeStruct((N, D), _f32),
    )


def kernel(x, edge_index, W, b):
    N, D_in = x.shape
    D_out = W.shape[0]
    E = edge_index.shape[1]

    ei = edge_index.astype(_i32).reshape(2 * E)

    deg_p = _deg_kernel(N, E)(ei)
    g = _prescale_kernel(N, D_in, D_out)(x, W, deg_p)
    p = _agg_kernel(N, D_out, E)(ei, g)
    out = _finish_kernel(N, D_out)(p, deg_p,
                                   b.astype(_f32).reshape(1, D_out))
    return out
